# baseline (speedup 1.0000x reference)
import functools

import jax
import jax.numpy as jnp
from jax import lax
from jax.experimental import pallas as pl
from jax.experimental.pallas import tpu as pltpu
from jax.experimental.pallas import tpu_sc as plsc

D_MODEL = 128
MAX_LEN = 512
PAD = 32
TAB_ROWS = MAX_LEN + PAD
CHUNK = 128
NBUF = 2


def kernel(x, pe):
    info = plsc.get_sparse_core_info()
    nc, ns, nl = info.num_cores, info.num_subcores, info.num_lanes
    nw = nc * ns
    b = x.shape[0]
    assert b % (nw * CHUNK * NBUF) == 0
    b_per_w = b // nw
    n_chunks = b_per_w // CHUNK
    groups = CHUNK // nl

    mesh = plsc.VectorSubcoreMesh(core_axis_name="c", subcore_axis_name="s")

    @functools.partial(
        pl.kernel,
        mesh=mesh,
        out_type=jax.ShapeDtypeStruct((b * D_MODEL,), jnp.float32),
        scratch_types=(
            [
                pltpu.VMEM((TAB_ROWS * D_MODEL,), jnp.float32),
                pltpu.VMEM((b_per_w,), jnp.int32),
            ]
            + [pltpu.VMEM((CHUNK * D_MODEL,), jnp.float32) for _ in range(NBUF)]
            + [pltpu.SemaphoreType.DMA for _ in range(NBUF + 2)]
        ),
        compiler_params=pltpu.CompilerParams(needs_layout_passes=False),
    )
    def sc_gather(idx_hbm, table_hbm, out_hbm, tab_v, idx_v, *bufs_and_sems):
        obuf = bufs_and_sems[:NBUF]
        ssem = bufs_and_sems[NBUF:2 * NBUF]
        isem, tsem = bufs_and_sems[2 * NBUF:]
        wid = lax.axis_index("s") * nc + lax.axis_index("c")
        base = wid * b_per_w

        pltpu.async_copy(idx_hbm.at[pl.ds(wid * b_per_w, b_per_w)], idx_v,
                         isem)
        pltpu.async_copy(table_hbm, tab_v.at[pl.ds(0, MAX_LEN * D_MODEL)],
                         tsem)
        pltpu.make_async_copy(table_hbm,
                              tab_v.at[pl.ds(0, MAX_LEN * D_MODEL)],
                              tsem).wait()

        for j in range(D_MODEL // nl):
            v = tab_v[pl.ds(j * nl, nl)]
            for r in range(PAD):
                tab_v[pl.ds((MAX_LEN + r) * D_MODEL + j * nl, nl)] = v

        pltpu.make_async_copy(idx_hbm.at[pl.ds(wid * b_per_w, b_per_w)],
                              idx_v, isem).wait()

        lane = lax.iota(jnp.int32, nl)
        obase0 = lane * D_MODEL

        def out_slice(ci):
            return out_hbm.at[pl.ds((base + ci * CHUNK) * D_MODEL,
                                    CHUNK * D_MODEL)]

        def fill_chunk(ci, bi):
            for g in range(groups):
                v = idx_v[pl.ds(ci * CHUNK + g * nl, nl)]
                rowidx = jnp.where(v < MAX_LEN,
                                   v, MAX_LEN + (v & (PAD - 1)))
                abase = rowidx * D_MODEL
                obase = obase0 + g * (nl * D_MODEL)

                def col(c, carry):
                    vals = plsc.load_gather(tab_v, [abase + c])
                    plsc.store_scatter(obuf[bi], [obase + c], vals)
                    return carry

                lax.fori_loop(0, D_MODEL, col, 0, unroll=8)

        for bi in range(NBUF):
            fill_chunk(bi, bi)
            pltpu.async_copy(obuf[bi], out_slice(bi), ssem[bi])

        def step(g, c):
            for bi in range(NBUF):
                ci = g * NBUF + bi + NBUF

                @pl.when(ci < n_chunks)
                def _():
                    pltpu.make_async_copy(
                        obuf[bi], out_slice(ci), ssem[bi]).wait()
                    fill_chunk(ci, bi)
                    pltpu.async_copy(obuf[bi], out_slice(ci), ssem[bi])

            return c

        lax.fori_loop(0, n_chunks // NBUF, step, 0)

        for bi in range(NBUF):
            pltpu.make_async_copy(obuf[bi], out_slice(0), ssem[bi]).wait()

    return sc_gather(x, pe.reshape(-1)).reshape(b, D_MODEL)


# speedup vs baseline: 20.9090x; 20.9090x over previous
import functools

import jax
import jax.numpy as jnp
from jax import lax
from jax.experimental import pallas as pl
from jax.experimental.pallas import tpu as pltpu
from jax.experimental.pallas import tpu_sc as plsc

D_MODEL = 128
MAX_LEN = 512
IDX_RANGE = 1024
CHUNK = 128
SLAB = 2
NBUF = 2
FILL = (IDX_RANGE - MAX_LEN) // 16


def kernel(x, pe):
    info = plsc.get_sparse_core_info()
    nc, ns, nl = info.num_cores, info.num_subcores, info.num_lanes
    nw = nc * ns
    b = x.shape[0]
    assert b % (nw * CHUNK * SLAB * NBUF) == 0
    b_per_w = b // nw
    n_chunks = b_per_w // CHUNK
    x2 = x.reshape(nw * n_chunks, CHUNK)

    mesh = plsc.VectorSubcoreMesh(core_axis_name="c", subcore_axis_name="s")

    @functools.partial(
        pl.kernel,
        mesh=mesh,
        out_type=jax.ShapeDtypeStruct((b, D_MODEL), jnp.float32),
        scratch_types=(
            [
                pltpu.VMEM((n_chunks, CHUNK), jnp.int32),
                pltpu.VMEM((D_MODEL,), jnp.float32),
                pltpu.VMEM((FILL, D_MODEL), jnp.float32),
                pltpu.MemorySpace.VMEM_SHARED((IDX_RANGE, D_MODEL), jnp.float32),
            ]
            + [pltpu.VMEM((SLAB * CHUNK, D_MODEL), jnp.float32)
               for _ in range(NBUF)]
            + [pltpu.SemaphoreType.DMA for _ in range(2 * NBUF + 2)]
        ),
    )
    def sc_gather(idx_hbm, table_hbm, out_hbm, idx_v, row0_v, fill_v, tab_sp,
                  *bufs_and_sems):
        rows = bufs_and_sems[:NBUF]
        gsem = bufs_and_sems[NBUF:2 * NBUF]
        ssem = bufs_and_sems[2 * NBUF:3 * NBUF]
        isem, tsem = bufs_and_sems[3 * NBUF:]
        sid = lax.axis_index("s")
        wid = sid * nc + lax.axis_index("c")
        base = wid * b_per_w

        pltpu.async_copy(idx_hbm.at[pl.ds(wid * n_chunks, n_chunks)], idx_v,
                         isem)

        @pl.when(sid == 0)
        def _():
            pltpu.async_copy(table_hbm, tab_sp.at[pl.ds(0, MAX_LEN)], tsem)

        pltpu.sync_copy(table_hbm.at[0], row0_v)
        for j in range(D_MODEL // nl):
            v = row0_v[pl.ds(j * nl, nl)]
            for r in range(FILL):
                fill_v[r, pl.ds(j * nl, nl)] = v
        pltpu.sync_copy(fill_v, tab_sp.at[pl.ds(MAX_LEN + sid * FILL, FILL)])

        @pl.when(sid == 0)
        def _():
            pltpu.make_async_copy(
                table_hbm, tab_sp.at[pl.ds(0, MAX_LEN)], tsem).wait()

        plsc.subcore_barrier()
        pltpu.make_async_copy(
            idx_hbm.at[pl.ds(wid * n_chunks, n_chunks)], idx_v, isem).wait()

        n_super = n_chunks // SLAB

        def out_slice(si):
            return out_hbm.at[pl.ds(base + si * SLAB * CHUNK, SLAB * CHUNK)]

        def gathers(si, bi):
            for k in range(SLAB):
                pltpu.async_copy(tab_sp.at[idx_v.at[si * SLAB + k]],
                                 rows[bi].at[pl.ds(k * CHUNK, CHUNK)],
                                 gsem[bi])

        def wait_gathers(si, bi):
            for k in range(SLAB):
                pltpu.make_async_copy(tab_sp.at[idx_v.at[si * SLAB + k]],
                                      rows[bi].at[pl.ds(k * CHUNK, CHUNK)],
                                      gsem[bi]).wait()

        for bi in range(NBUF):
            gathers(bi, bi)

        def step(g, c):
            for bi in range(NBUF):
                si = g * NBUF + bi
                nsi = si + NBUF
                wait_gathers(si, bi)
                pltpu.async_copy(rows[bi], out_slice(si), ssem[bi])

                @pl.when(nsi < n_super)
                def _():
                    pltpu.make_async_copy(
                        rows[bi], out_slice(si), ssem[bi]).wait()
                    gathers(nsi, bi)

            return c

        lax.fori_loop(0, n_super // NBUF, step, 0)

        for bi in range(NBUF):
            pltpu.make_async_copy(rows[bi], out_slice(0), ssem[bi]).wait()

    return sc_gather(x2, pe)


# speedup vs baseline: 21.3011x; 1.0188x over previous
import functools

import jax
import jax.numpy as jnp
from jax import lax
from jax.experimental import pallas as pl
from jax.experimental.pallas import tpu as pltpu
from jax.experimental.pallas import tpu_sc as plsc

D_MODEL = 128
MAX_LEN = 512
IDX_RANGE = 1024
CHUNK = 128
NBUF = 5
FILL = (IDX_RANGE - MAX_LEN) // 16


def kernel(x, pe):
    info = plsc.get_sparse_core_info()
    nc, ns, nl = info.num_cores, info.num_subcores, info.num_lanes
    nw = nc * ns
    b = x.shape[0]
    assert b % (nw * CHUNK * NBUF) == 0
    b_per_w = b // nw
    n_chunks = b_per_w // CHUNK
    x2 = x.reshape(nw * n_chunks, CHUNK)

    mesh = plsc.VectorSubcoreMesh(core_axis_name="c", subcore_axis_name="s")

    @functools.partial(
        pl.kernel,
        mesh=mesh,
        out_type=jax.ShapeDtypeStruct((b, D_MODEL), jnp.float32),
        scratch_types=(
            [
                pltpu.VMEM((n_chunks, CHUNK), jnp.int32),
                pltpu.VMEM((D_MODEL,), jnp.float32),
                pltpu.VMEM((FILL, D_MODEL), jnp.float32),
                pltpu.MemorySpace.VMEM_SHARED((IDX_RANGE, D_MODEL), jnp.float32),
            ]
            + [pltpu.VMEM((CHUNK, D_MODEL), jnp.float32) for _ in range(NBUF)]
            + [pltpu.SemaphoreType.DMA for _ in range(2 * NBUF + 2)]
        ),
    )
    def sc_gather(idx_hbm, table_hbm, out_hbm, idx_v, row0_v, fill_v, tab_sp,
                  *bufs_and_sems):
        rows = bufs_and_sems[:NBUF]
        gsem = bufs_and_sems[NBUF:2 * NBUF]
        ssem = bufs_and_sems[2 * NBUF:3 * NBUF]
        isem, tsem = bufs_and_sems[3 * NBUF:]
        sid = lax.axis_index("s")
        wid = sid * nc + lax.axis_index("c")
        base = wid * b_per_w

        pltpu.async_copy(idx_hbm.at[pl.ds(wid * n_chunks, n_chunks)], idx_v,
                         isem)

        @pl.when(sid == 0)
        def _():
            pltpu.async_copy(table_hbm, tab_sp.at[pl.ds(0, MAX_LEN)], tsem)

        pltpu.sync_copy(table_hbm.at[0], row0_v)
        for j in range(D_MODEL // nl):
            v = row0_v[pl.ds(j * nl, nl)]
            for r in range(FILL):
                fill_v[r, pl.ds(j * nl, nl)] = v
        pltpu.sync_copy(fill_v, tab_sp.at[pl.ds(MAX_LEN + sid * FILL, FILL)])

        @pl.when(sid == 0)
        def _():
            pltpu.make_async_copy(
                table_hbm, tab_sp.at[pl.ds(0, MAX_LEN)], tsem).wait()

        plsc.subcore_barrier()
        pltpu.make_async_copy(
            idx_hbm.at[pl.ds(wid * n_chunks, n_chunks)], idx_v, isem).wait()

        def out_slice(ci):
            return out_hbm.at[pl.ds(base + ci * CHUNK, CHUNK)]

        for bi in range(NBUF):
            pltpu.async_copy(tab_sp.at[idx_v.at[bi]], rows[bi], gsem[bi])

        def step(g, c):
            for bi in range(NBUF):
                ci = g * NBUF + bi
                nci = ci + NBUF
                pltpu.make_async_copy(
                    tab_sp.at[idx_v.at[ci]], rows[bi], gsem[bi]).wait()
                pltpu.async_copy(rows[bi], out_slice(ci), ssem[bi])

                @pl.when(nci < n_chunks)
                def _():
                    pltpu.make_async_copy(
                        rows[bi], out_slice(ci), ssem[bi]).wait()
                    pltpu.async_copy(
                        tab_sp.at[idx_v.at[nci]], rows[bi], gsem[bi])

            return c

        lax.fori_loop(0, n_chunks // NBUF, step, 0)

        for bi in range(NBUF):
            pltpu.make_async_copy(rows[bi], out_slice(0), ssem[bi]).wait()

    return sc_gather(x2, pe)
